# Initial kernel scaffold; baseline (speedup 1.0000x reference)
import jax
import jax.numpy as jnp
from jax.experimental import pallas as pl


def kernel(x, edge_index, batch_index, gcn_w, gcn_b, topk_w, cheb_w0, cheb_b0, cheb_w1, cheb_b1, cheb_w2, cheb_b2, bn_g0, bn_b0, bn_g1, bn_b1, bn_g2, bn_b2, lin0_w, lin0_b, lin1_w, lin1_b, linf_w, linf_b):
    raise NotImplementedError("write your pallas kernel here")


# speedup vs baseline: 2.5768x; 2.5768x over previous
import functools

import jax
import jax.numpy as jnp
import numpy as np
from jax.experimental import pallas as pl
from jax.experimental.pallas import tpu as pltpu

N = 50000
E = 800000
B = 128
GW = 128
MIN_SCORE = 0.1
KCAP = 2048
ECAP = 4096
NEG = -3.0e38


def _elu(v):
    return jnp.where(v > 0, v, jnp.exp(jnp.minimum(v, 0.0)) - 1.0)


def _a8(i):
    return pl.multiple_of(i * 8, 8)


def _cheb_body(xc_ref, cb_ref, cs_ref, cd_ref, cn_ref, kn_ref, ke_ref,
               w00, w01, b0, g0, bb0, w10, w11, b1, g1, bb1,
               w20, w21, b2, g2, bb2, l0w, l0b, l1w, l1b, lfw, lfb,
               out_ref, ha, hb, zbuf):
    bnc = np.float32(1.0 / np.sqrt(1.0 + 1e-5))
    kn = kn_ref[0]
    ke = ke_ref[0]
    nblk = (kn * 8 + 255) // 256

    ha[...] = xc_ref[...]
    zbuf[...] = jnp.full((B * 8, 3 * GW), NEG, jnp.float32)

    for i, (w0, w1, cb, gg, gb) in enumerate(
        [(w00, w01, b0, g0, bb0), (w10, w11, b1, g1, bb1),
         (w20, w21, b2, g2, bb2)]):
        cur = ha if i % 2 == 0 else hb
        nxt = hb if i % 2 == 0 else ha
        nxt[...] = jnp.zeros((KCAP * 8, GW), jnp.float32)

        def ebody(e, _):
            s = cs_ref[e]
            d = cd_ref[e]
            w = cn_ref[e]
            blk = cur[pl.ds(_a8(s), 8), :] * w
            nxt[pl.ds(_a8(d), 8), :] = nxt[pl.ds(_a8(d), 8), :] + blk
            return 0

        jax.lax.fori_loop(0, ke, ebody, 0)

        def mblk(j, _):
            r = pl.multiple_of(j * 256, 256)
            hj = cur[pl.ds(r, 256), :]
            tj = nxt[pl.ds(r, 256), :]
            t = jnp.dot(hj, w0[...], preferred_element_type=jnp.float32)
            t = t + jnp.dot(tj, w1[...], preferred_element_type=jnp.float32)
            t = _elu(t + cb[...]) * bnc * gg[...] + gb[...]
            nxt[pl.ds(r, 256), :] = t
            return 0

        jax.lax.fori_loop(0, nblk, mblk, 0)

        col = i * GW

        def pbody(n, _):
            b = cb_ref[n]
            blk = nxt[pl.ds(_a8(n), 8), :]
            old = zbuf[pl.ds(_a8(b), 8), pl.ds(col, GW)]
            zbuf[pl.ds(_a8(b), 8), pl.ds(col, GW)] = jnp.maximum(old, blk)
            return 0

        jax.lax.fori_loop(0, kn, pbody, 0)

    z = zbuf[...]
    z = jnp.where(z > NEG * 0.5, z, 0.0)
    z = _elu(jnp.dot(z, l0w[...], preferred_element_type=jnp.float32) + l0b[...])
    z = _elu(jnp.dot(z, l1w[...], preferred_element_type=jnp.float32) + l1b[...])
    z = jnp.dot(z, lfw[...], preferred_element_type=jnp.float32) + lfb[...]
    cmask = jax.lax.broadcasted_iota(jnp.int32, (B * 8, 128), 1) < 3
    zm = jnp.where(cmask, z, NEG)
    m = jnp.max(zm, axis=1, keepdims=True)
    s = jnp.sum(jnp.where(cmask, jnp.exp(zm - m), 0.0), axis=1, keepdims=True)
    out_ref[...] = z - m - jnp.log(s)


def _cheb_pallas(xcpad, cbatch, csrc, cdst, cnorm, kn, ke, wlist):
    smem = functools.partial(pl.BlockSpec, memory_space=pltpu.SMEM)
    vmem = functools.partial(pl.BlockSpec, memory_space=pltpu.VMEM)
    in_specs = ([vmem(), smem(), smem(), smem(), smem(), smem(), smem()]
                + [vmem() for _ in wlist])
    return pl.pallas_call(
        _cheb_body,
        out_shape=jax.ShapeDtypeStruct((B * 8, 128), jnp.float32),
        in_specs=in_specs,
        out_specs=vmem(),
        scratch_shapes=[
            pltpu.VMEM((KCAP * 8, GW), jnp.float32),
            pltpu.VMEM((KCAP * 8, GW), jnp.float32),
            pltpu.VMEM((B * 8, 3 * GW), jnp.float32),
        ],
    )(xcpad, cbatch, csrc, cdst, cnorm, kn, ke, *wlist)


def _cheb_dense_fallback(x2, src, dst, emask, batch_index, mask, cws, cbs,
                         bgs, bbs):
    bnc = np.float32(1.0 / np.sqrt(1.0 + 1e-5))
    deg = jax.ops.segment_sum(emask, dst, num_segments=N)
    dinv = jnp.where(deg > 0, deg ** -0.5, 0.0)
    norm = -(dinv[src] * dinv[dst]) * emask
    h = x2
    pools = []
    for i in range(3):
        tx1 = jax.ops.segment_sum(h[src] * norm[:, None], dst, num_segments=N)
        h = jax.nn.elu(h @ cws[i][0] + tx1 @ cws[i][1] + cbs[i])
        h = h * bnc * bgs[i] + bbs[i]
        g = jax.ops.segment_max(jnp.where(mask[:, None], h, -jnp.inf),
                                batch_index, num_segments=B)
        pools.append(jnp.where(jnp.isfinite(g), g, 0.0))
    return jnp.concatenate(pools, axis=1)


def kernel(x, edge_index, batch_index, gcn_w, gcn_b, topk_w, cheb_w0, cheb_b0, cheb_w1, cheb_b1, cheb_w2, cheb_b2, bn_g0, bn_b0, bn_g1, bn_b1, bn_g2, bn_b2, lin0_w, lin0_b, lin1_w, lin1_b, linf_w, linf_b):
    src, dst = edge_index[0], edge_index[1]

    deg = jax.ops.segment_sum(jnp.ones((E,), jnp.float32), dst, num_segments=N) + 1.0
    dinv = deg ** -0.5
    h1d = x[:, 0] * gcn_w[0, 0] + x[:, 1] * gcn_w[1, 0]
    u = h1d * dinv
    attnum = jax.ops.segment_sum(u[src], dst, num_segments=N)
    attn = dinv * (attnum + u) + gcn_b[0]

    logit = attn * topk_w[0]
    m = jax.ops.segment_max(logit, batch_index, num_segments=B)
    e = jnp.exp(logit - m[batch_index])
    z = jax.ops.segment_sum(e, batch_index, num_segments=B)
    score = e / z[batch_index]
    thr = jnp.minimum(1.0 / z - 1e-7, MIN_SCORE)
    mask = score > thr[batch_index]

    maskf = mask.astype(jnp.float32)
    em = maskf[src] * maskf[dst]
    deg2 = jax.ops.segment_sum(em, dst, num_segments=N)
    dinv2 = jnp.where(deg2 > 0, deg2 ** -0.5, 0.0)
    x2 = jnp.where(mask[:, None], x * score[:, None], 0.0)

    kn = mask.sum().astype(jnp.int32)
    ek = (em > 0).sum().astype(jnp.int32)
    nidx = jnp.nonzero(mask, size=KCAP, fill_value=N - 1)[0]
    node_pos = jnp.cumsum(maskf).astype(jnp.int32) - 1
    eidx = jnp.nonzero(em > 0, size=ECAP, fill_value=E - 1)[0]
    asrc, adst = src[eidx], dst[eidx]
    valid_n = jnp.arange(KCAP, dtype=jnp.int32) < kn
    valid_e = jnp.arange(ECAP, dtype=jnp.int32) < ek
    csrc = jnp.where(valid_e, node_pos[asrc], 0)
    cdst = jnp.where(valid_e, node_pos[adst], 0)
    cnorm = jnp.where(valid_e, -(dinv2[asrc] * dinv2[adst]), 0.0)
    xc = jnp.where(valid_n[:, None], x2[nidx], 0.0)
    xcr = jnp.repeat(xc, 8, axis=0)
    xcpad = jnp.zeros((KCAP * 8, GW), jnp.float32).at[:, :2].set(xcr)
    cbatch = jnp.where(valid_n, batch_index[nidx], B - 1)

    def padw(w, rows=GW, cols=GW):
        out = jnp.zeros((rows, cols), jnp.float32)
        return out.at[:w.shape[0], :w.shape[1]].set(w)

    def padb(b, n=GW):
        return jnp.zeros((n,), jnp.float32).at[:b.shape[0]].set(b)

    wlist = [
        padw(cheb_w0[0]), padw(cheb_w0[1]), cheb_b0, bn_g0, bn_b0,
        padw(cheb_w1[0]), padw(cheb_w1[1]), cheb_b1, bn_g1, bn_b1,
        padw(cheb_w2[0]), padw(cheb_w2[1]), cheb_b2, bn_g2, bn_b2,
        padw(lin0_w, 3 * GW, 128), padb(lin0_b, 128),
        padw(lin1_w, 128, 128), padb(lin1_b, 128),
        padw(linf_w, 128, 128), padb(linf_b, 128),
    ]

    def fast_path(_):
        out = _cheb_pallas(xcpad, cbatch, csrc, cdst, cnorm,
                           kn[None], ek[None], wlist)
        return out[::8, :3]

    def slow_path(_):
        zc = _cheb_dense_fallback(x2, src, dst, em, batch_index, mask,
                                  [cheb_w0, cheb_w1, cheb_w2],
                                  [cheb_b0, cheb_b1, cheb_b2],
                                  [bn_g0, bn_g1, bn_g2],
                                  [bn_b0, bn_b1, bn_b2])
        zc = jax.nn.elu(zc @ lin0_w + lin0_b)
        zc = jax.nn.elu(zc @ lin1_w + lin1_b)
        zc = zc @ linf_w + linf_b
        return jax.nn.log_softmax(zc, axis=1)

    ok = (kn <= KCAP) & (ek <= ECAP)
    return jax.lax.cond(ok, fast_path, slow_path, 0)


# speedup vs baseline: 63.8427x; 24.7759x over previous
import functools

import jax
import jax.numpy as jnp
import numpy as np
from jax import lax
from jax.experimental import pallas as pl
from jax.experimental.pallas import tpu as pltpu
from jax.experimental.pallas import tpu_sc as plsc

N = 50000
E = 800000
B = 128
GW = 128
MIN_SCORE = 0.1
KCAP = 2048
ECAP = 4096
NPAD = 50176
NROW = NPAD // 128
NSUB = NPAD // 16
ECH = 8000
NCHUNK = E // ECH
NEG = np.float32(-3.0e38)
POS = np.float32(3.0e38)


def _fill_vmem(ref, n, val):
    def body(i, _):
        ref[pl.ds(i * 16, 16)] = jnp.full((16,), val, ref.dtype)
        return 0
    lax.fori_loop(0, n // 16, body, 0)


def _sc_mesh():
    return plsc.VectorSubcoreMesh(core_axis_name="c", subcore_axis_name="s")


def _sub8(i):
    return pl.multiple_of(i, 8)


def sc_deg(dst):
    @functools.partial(
        pl.kernel, mesh=_sc_mesh(),
        out_type=jax.ShapeDtypeStruct((2 * NPAD,), jnp.float32),
        scratch_types=[
            pltpu.VMEM_SHARED((NPAD,), jnp.float32),
            pltpu.VMEM((NSUB,), jnp.float32),
            pltpu.VMEM((ECH,), jnp.int32),
            pltpu.VMEM((ECH,), jnp.float32),
        ],
    )
    def k(dst_hbm, out_hbm, acc, zbuf, idxv, onev):
        cid = lax.axis_index("c")
        sid = lax.axis_index("s")
        tid = cid * 16 + sid
        _fill_vmem(zbuf, NSUB, 0.0)
        pltpu.sync_copy(zbuf, acc.at[pl.ds(_sub8(sid * NSUB), NSUB)])
        _fill_vmem(onev, ECH, 1.0)
        plsc.subcore_barrier()

        def chunk(j, _):
            ci = tid + j * 32

            @pl.when(ci < NCHUNK)
            def _():
                pltpu.sync_copy(dst_hbm.at[pl.ds(_sub8(ci * ECH), ECH)], idxv)
                pltpu.sync_copy(onev, acc.at[idxv], add=True)
            return 0

        lax.fori_loop(0, (NCHUNK + 31) // 32, chunk, 0)
        plsc.subcore_barrier()
        pltpu.sync_copy(acc.at[pl.ds(_sub8(sid * NSUB), NSUB)], zbuf)
        pltpu.sync_copy(
            zbuf, out_hbm.at[pl.ds(_sub8(cid * NPAD + sid * NSUB), NSUB)])

    return k(dst)


def sc_gather_scatter(src, dst, vals):
    @functools.partial(
        pl.kernel, mesh=_sc_mesh(),
        out_type=jax.ShapeDtypeStruct((2 * NPAD,), jnp.float32),
        scratch_types=[
            pltpu.VMEM_SHARED((NPAD,), jnp.float32),
            pltpu.VMEM_SHARED((NPAD,), jnp.float32),
            pltpu.VMEM((NSUB,), jnp.float32),
            pltpu.VMEM((ECH,), jnp.int32),
            pltpu.VMEM((ECH,), jnp.float32),
        ],
    )
    def k(src_hbm, dst_hbm, vals_hbm, out_hbm, ush, acc, zbuf, idxv, valv):
        cid = lax.axis_index("c")
        sid = lax.axis_index("s")
        tid = cid * 16 + sid
        _fill_vmem(zbuf, NSUB, 0.0)
        pltpu.sync_copy(zbuf, acc.at[pl.ds(_sub8(sid * NSUB), NSUB)])
        pltpu.sync_copy(vals_hbm.at[pl.ds(_sub8(sid * NSUB), NSUB)], zbuf)
        pltpu.sync_copy(zbuf, ush.at[pl.ds(_sub8(sid * NSUB), NSUB)])
        plsc.subcore_barrier()

        def chunk(j, _):
            ci = tid + j * 32

            @pl.when(ci < NCHUNK)
            def _():
                off = _sub8(ci * ECH)
                pltpu.sync_copy(src_hbm.at[pl.ds(off, ECH)], idxv)
                pltpu.sync_copy(ush.at[idxv], valv)
                pltpu.sync_copy(dst_hbm.at[pl.ds(off, ECH)], idxv)
                pltpu.sync_copy(valv, acc.at[idxv], add=True)
            return 0

        lax.fori_loop(0, (NCHUNK + 31) // 32, chunk, 0)
        plsc.subcore_barrier()
        pltpu.sync_copy(acc.at[pl.ds(_sub8(sid * NSUB), NSUB)], zbuf)
        pltpu.sync_copy(
            zbuf, out_hbm.at[pl.ds(_sub8(cid * NPAD + sid * NSUB), NSUB)])

    return k(src, dst, vals)


def sc_emask(src, dst, maskf):
    @functools.partial(
        pl.kernel, mesh=_sc_mesh(),
        out_type=jax.ShapeDtypeStruct((E + 2 * NPAD,), jnp.float32),
        scratch_types=[
            pltpu.VMEM_SHARED((NPAD,), jnp.float32),
            pltpu.VMEM_SHARED((NPAD,), jnp.float32),
            pltpu.VMEM((NSUB,), jnp.float32),
            pltpu.VMEM((ECH,), jnp.int32),
            pltpu.VMEM((ECH,), jnp.float32),
            pltpu.VMEM((ECH,), jnp.float32),
        ],
    )
    def k(src_hbm, dst_hbm, m_hbm, out_hbm, msh, acc, zbuf, idxv, va, vb):
        cid = lax.axis_index("c")
        sid = lax.axis_index("s")
        tid = cid * 16 + sid
        _fill_vmem(zbuf, NSUB, 0.0)
        pltpu.sync_copy(zbuf, acc.at[pl.ds(_sub8(sid * NSUB), NSUB)])
        pltpu.sync_copy(m_hbm.at[pl.ds(_sub8(sid * NSUB), NSUB)], zbuf)
        pltpu.sync_copy(zbuf, msh.at[pl.ds(_sub8(sid * NSUB), NSUB)])
        plsc.subcore_barrier()

        def chunk(j, _):
            ci = tid + j * 32

            @pl.when(ci < NCHUNK)
            def _():
                off = _sub8(ci * ECH)
                pltpu.sync_copy(src_hbm.at[pl.ds(off, ECH)], idxv)
                pltpu.sync_copy(msh.at[idxv], va)
                pltpu.sync_copy(dst_hbm.at[pl.ds(off, ECH)], idxv)
                pltpu.sync_copy(msh.at[idxv], vb)

                def mul(i, _):
                    sl = pl.ds(i * 16, 16)
                    vb[sl] = va[sl] * vb[sl]
                    return 0

                lax.fori_loop(0, ECH // 16, mul, 0)
                pltpu.sync_copy(vb, out_hbm.at[pl.ds(off, ECH)])
                pltpu.sync_copy(vb, acc.at[idxv], add=True)
            return 0

        lax.fori_loop(0, (NCHUNK + 31) // 32, chunk, 0)
        plsc.subcore_barrier()
        pltpu.sync_copy(acc.at[pl.ds(_sub8(sid * NSUB), NSUB)], zbuf)
        pltpu.sync_copy(
            zbuf, out_hbm.at[pl.ds(_sub8(E + cid * NPAD + sid * NSUB), NSUB)])

    return k(src, dst, maskf)


def _elu(v):
    return jnp.where(v > 0, v, jnp.exp(jnp.minimum(v, 0.0)) - 1.0)


def _tc_prep_body(xa, xb, dp, w_ref, u_ref, dinv_ref):
    deg = dp[0] + dp[1] + 1.0
    dinv = lax.rsqrt(deg)
    h1d = xa[...] * w_ref[0] + xb[...] * w_ref[1]
    u_ref[...] = h1d * dinv
    dinv_ref[...] = dinv


def tc_prep(xa, xb, degp, w2):
    vmem = functools.partial(pl.BlockSpec, memory_space=pltpu.VMEM)
    smem = functools.partial(pl.BlockSpec, memory_space=pltpu.SMEM)
    return pl.pallas_call(
        _tc_prep_body,
        out_shape=(jax.ShapeDtypeStruct((NROW, 128), jnp.float32),
                   jax.ShapeDtypeStruct((NROW, 128), jnp.float32)),
        in_specs=[vmem(), vmem(), vmem(), smem()],
        out_specs=(vmem(), vmem()),
    )(xa, xb, degp, w2)


def _tc_softmax_body(ap, u, dinv, batch, xa, xb, sc_ref,
                     mf_ref, x2a_ref, x2b_ref, sm_ref):
    tw = sc_ref[0]
    gb = sc_ref[1]
    attn = dinv[...] * (ap[0] + ap[1] + u[...]) + gb
    logit = attn * tw
    bt = batch[...]

    def mloop(b, mb_bcast):
        mb = jnp.max(jnp.where(bt == b, logit, NEG))
        return jnp.where(bt == b, mb, mb_bcast)

    mb_bcast = lax.fori_loop(0, B, mloop,
                             jnp.full((NROW, 128), POS, jnp.float32))
    e = jnp.exp(logit - mb_bcast)

    def zloop(b, carry):
        z_bcast, thr_bcast = carry
        zb = jnp.sum(jnp.where(bt == b, e, 0.0))
        zb = jnp.maximum(zb, 1e-37)
        thrb = jnp.minimum(1.0 / zb - 1e-7, MIN_SCORE)
        return (jnp.where(bt == b, zb, z_bcast),
                jnp.where(bt == b, thrb, thr_bcast))

    z_bcast, thr_bcast = lax.fori_loop(
        0, B, zloop, (jnp.ones((NROW, 128), jnp.float32),
                      jnp.full((NROW, 128), POS, jnp.float32)))
    score = e / z_bcast
    maskf = jnp.where(score > thr_bcast, 1.0, 0.0)
    mf_ref[...] = maskf
    sm = score * maskf
    sm_ref[...] = sm
    x2a_ref[...] = xa[...] * sm
    x2b_ref[...] = xb[...] * sm


def tc_softmax(attp, u, dinv, batchp, xa, xb, scalars):
    vmem = functools.partial(pl.BlockSpec, memory_space=pltpu.VMEM)
    smem = functools.partial(pl.BlockSpec, memory_space=pltpu.SMEM)
    return pl.pallas_call(
        _tc_softmax_body,
        out_shape=(jax.ShapeDtypeStruct((NROW, 128), jnp.float32),
                   jax.ShapeDtypeStruct((NROW, 128), jnp.float32),
                   jax.ShapeDtypeStruct((NROW, 128), jnp.float32),
                   jax.ShapeDtypeStruct((NROW, 128), jnp.float32)),
        in_specs=[vmem(), vmem(), vmem(), vmem(), vmem(), vmem(), smem()],
        out_specs=(vmem(), vmem(), vmem(), vmem()),
    )(attp, u, dinv, batchp, xa, xb, scalars)


def _a8(i):
    return pl.multiple_of(i * 8, 8)


def _cheb_body(xc_ref, cb_ref, cs_ref, cd_ref, cn_ref, kn_ref, ke_ref,
               w00, w01, b0, g0, bb0, w10, w11, b1, g1, bb1,
               w20, w21, b2, g2, bb2, l0w, l0b, l1w, l1b, lfw, lfb,
               out_ref, ha, hb, zbuf):
    bnc = np.float32(1.0 / np.sqrt(1.0 + 1e-5))
    kn = kn_ref[0]
    ke = ke_ref[0]
    nblk = (kn * 8 + 255) // 256

    ha[...] = xc_ref[...]
    zbuf[...] = jnp.full((B * 8, 3 * GW), NEG, jnp.float32)

    for i, (w0, w1, cb, gg, gb) in enumerate(
        [(w00, w01, b0, g0, bb0), (w10, w11, b1, g1, bb1),
         (w20, w21, b2, g2, bb2)]):
        cur = ha if i % 2 == 0 else hb
        nxt = hb if i % 2 == 0 else ha
        nxt[...] = jnp.zeros((KCAP * 8, GW), jnp.float32)

        def ebody(e, _):
            s = cs_ref[e]
            d = cd_ref[e]
            w = cn_ref[e]
            blk = cur[pl.ds(_a8(s), 8), :] * w
            nxt[pl.ds(_a8(d), 8), :] = nxt[pl.ds(_a8(d), 8), :] + blk
            return 0

        jax.lax.fori_loop(0, ke, ebody, 0)

        def mblk(j, _):
            r = pl.multiple_of(j * 256, 256)
            hj = cur[pl.ds(r, 256), :]
            tj = nxt[pl.ds(r, 256), :]
            t = jnp.dot(hj, w0[...], preferred_element_type=jnp.float32)
            t = t + jnp.dot(tj, w1[...], preferred_element_type=jnp.float32)
            t = _elu(t + cb[...]) * bnc * gg[...] + gb[...]
            nxt[pl.ds(r, 256), :] = t
            return 0

        jax.lax.fori_loop(0, nblk, mblk, 0)

        col = i * GW

        def pbody(n, _):
            b = cb_ref[n]
            blk = nxt[pl.ds(_a8(n), 8), :]
            old = zbuf[pl.ds(_a8(b), 8), pl.ds(col, GW)]
            zbuf[pl.ds(_a8(b), 8), pl.ds(col, GW)] = jnp.maximum(old, blk)
            return 0

        jax.lax.fori_loop(0, kn, pbody, 0)

    z = zbuf[...]
    z = jnp.where(z > NEG * 0.5, z, 0.0)
    z = _elu(jnp.dot(z, l0w[...], preferred_element_type=jnp.float32) + l0b[...])
    z = _elu(jnp.dot(z, l1w[...], preferred_element_type=jnp.float32) + l1b[...])
    z = jnp.dot(z, lfw[...], preferred_element_type=jnp.float32) + lfb[...]
    cmask = jax.lax.broadcasted_iota(jnp.int32, (B * 8, 128), 1) < 3
    zm = jnp.where(cmask, z, NEG)
    m = jnp.max(zm, axis=1, keepdims=True)
    s = jnp.sum(jnp.where(cmask, jnp.exp(zm - m), 0.0), axis=1, keepdims=True)
    out_ref[...] = z - m - jnp.log(s)


def _cheb_pallas(xcpad, cbatch, csrc, cdst, cnorm, kn, ke, wlist):
    smem = functools.partial(pl.BlockSpec, memory_space=pltpu.SMEM)
    vmem = functools.partial(pl.BlockSpec, memory_space=pltpu.VMEM)
    in_specs = ([vmem(), smem(), smem(), smem(), smem(), smem(), smem()]
                + [vmem() for _ in wlist])
    return pl.pallas_call(
        _cheb_body,
        out_shape=jax.ShapeDtypeStruct((B * 8, 128), jnp.float32),
        in_specs=in_specs,
        out_specs=vmem(),
        scratch_shapes=[
            pltpu.VMEM((KCAP * 8, GW), jnp.float32),
            pltpu.VMEM((KCAP * 8, GW), jnp.float32),
            pltpu.VMEM((B * 8, 3 * GW), jnp.float32),
        ],
    )(xcpad, cbatch, csrc, cdst, cnorm, kn, ke, *wlist)


def _cheb_dense_fallback(x2, src, dst, emask, batch_index, mask, cws, cbs,
                         bgs, bbs):
    bnc = np.float32(1.0 / np.sqrt(1.0 + 1e-5))
    deg = jax.ops.segment_sum(emask, dst, num_segments=N)
    dinv = jnp.where(deg > 0, deg ** -0.5, 0.0)
    norm = -(dinv[src] * dinv[dst]) * emask
    h = x2
    pools = []
    for i in range(3):
        tx1 = jax.ops.segment_sum(h[src] * norm[:, None], dst, num_segments=N)
        h = jax.nn.elu(h @ cws[i][0] + tx1 @ cws[i][1] + cbs[i])
        h = h * bnc * bgs[i] + bbs[i]
        g = jax.ops.segment_max(jnp.where(mask[:, None], h, -jnp.inf),
                                batch_index, num_segments=B)
        pools.append(jnp.where(jnp.isfinite(g), g, 0.0))
    return jnp.concatenate(pools, axis=1)


def kernel(x, edge_index, batch_index, gcn_w, gcn_b, topk_w, cheb_w0, cheb_b0, cheb_w1, cheb_b1, cheb_w2, cheb_b2, bn_g0, bn_b0, bn_g1, bn_b1, bn_g2, bn_b2, lin0_w, lin0_b, lin1_w, lin1_b, linf_w, linf_b):
    src, dst = edge_index[0], edge_index[1]

    def padn(v, fill):
        return jnp.full((NPAD,), fill, v.dtype).at[:N].set(v)

    xa2 = padn(x[:, 0], 0.0).reshape(NROW, 128)
    xb2 = padn(x[:, 1], 0.0).reshape(NROW, 128)
    batchp = padn(batch_index, 999).reshape(NROW, 128)

    degp = sc_deg(dst)
    u2d, dinv2d = tc_prep(xa2, xb2, degp.reshape(2, NROW, 128),
                          jnp.stack([gcn_w[0, 0], gcn_w[1, 0]]))
    attp = sc_gather_scatter(src, dst, u2d.reshape(NPAD))
    mf2d, x2a2, x2b2, sm2d = tc_softmax(
        attp.reshape(2, NROW, 128), u2d, dinv2d, batchp, xa2, xb2,
        jnp.stack([topk_w[0], gcn_b[0]]))
    maskf = mf2d.reshape(NPAD)[:N]
    emf = sc_emask(src, dst, mf2d.reshape(NPAD))
    em = emf[:E]
    deg2p = emf[E:].reshape(2, NPAD)
    deg2 = (deg2p[0] + deg2p[1])[:N]
    dinv2 = jnp.where(deg2 > 0, lax.rsqrt(deg2), 0.0)
    x2a = x2a2.reshape(NPAD)[:N]
    x2b = x2b2.reshape(NPAD)[:N]
    mask = maskf > 0

    kn = maskf.sum().astype(jnp.int32)
    ek = (em > 0).sum().astype(jnp.int32)
    nidx = jnp.nonzero(mask, size=KCAP, fill_value=N - 1)[0]
    node_pos = jnp.cumsum(maskf).astype(jnp.int32) - 1
    eidx = jnp.nonzero(em > 0, size=ECAP, fill_value=E - 1)[0]
    asrc, adst = src[eidx], dst[eidx]
    valid_n = jnp.arange(KCAP, dtype=jnp.int32) < kn
    valid_e = jnp.arange(ECAP, dtype=jnp.int32) < ek
    csrc = jnp.where(valid_e, node_pos[asrc], 0)
    cdst = jnp.where(valid_e, node_pos[adst], 0)
    cnorm = jnp.where(valid_e, -(dinv2[asrc] * dinv2[adst]), 0.0)
    xc = jnp.where(valid_n[:, None],
                   jnp.stack([x2a[nidx], x2b[nidx]], axis=1), 0.0)
    xcr = jnp.repeat(xc, 8, axis=0)
    xcpad = jnp.zeros((KCAP * 8, GW), jnp.float32).at[:, :2].set(xcr)
    cbatch = jnp.where(valid_n, batch_index[nidx], B - 1)

    def padw(w, rows=GW, cols=GW):
        out = jnp.zeros((rows, cols), jnp.float32)
        return out.at[:w.shape[0], :w.shape[1]].set(w)

    def padb(b, n=GW):
        return jnp.zeros((n,), jnp.float32).at[:b.shape[0]].set(b)

    wlist = [
        padw(cheb_w0[0]), padw(cheb_w0[1]), cheb_b0, bn_g0, bn_b0,
        padw(cheb_w1[0]), padw(cheb_w1[1]), cheb_b1, bn_g1, bn_b1,
        padw(cheb_w2[0]), padw(cheb_w2[1]), cheb_b2, bn_g2, bn_b2,
        padw(lin0_w, 3 * GW, 128), padb(lin0_b, 128),
        padw(lin1_w, 128, 128), padb(lin1_b, 128),
        padw(linf_w, 128, 128), padb(linf_b, 128),
    ]

    def fast_path(_):
        out = _cheb_pallas(xcpad, cbatch, csrc, cdst, cnorm,
                           kn[None], ek[None], wlist)
        return out[::8, :3]

    def slow_path(_):
        x2 = jnp.stack([x2a, x2b], axis=1)
        zc = _cheb_dense_fallback(x2, src, dst, em, batch_index, mask,
                                  [cheb_w0, cheb_w1, cheb_w2],
                                  [cheb_b0, cheb_b1, cheb_b2],
                                  [bn_g0, bn_g1, bn_g2],
                                  [bn_b0, bn_b1, bn_b2])
        zc = jax.nn.elu(zc @ lin0_w + lin0_b)
        zc = jax.nn.elu(zc @ lin1_w + lin1_b)
        zc = zc @ linf_w + linf_b
        return jax.nn.log_softmax(zc, axis=1)

    ok = (kn <= KCAP) & (ek <= ECAP)
    return jax.lax.cond(ok, fast_path, slow_path, 0)
